# baseline (speedup 1.0000x reference)
import functools

import jax
import jax.numpy as jnp
from jax.experimental import pallas as pl
from jax.experimental.pallas import tpu as pltpu
from jax.experimental.pallas import tpu_sc as plsc

B, T, N, K = 2, 8, 1024, 30
PC_DIM, FS_DIM = 3, 12
IN_CH = PC_DIM + FS_DIM
S = B * T
NTOT = S * N
NE = NTOT * K
EPS = 1e-5

_BF = jnp.bfloat16
_NB = 512
_GW = 512


def _dot(a, b):
    return jnp.dot(a.astype(_BF), b.astype(_BF),
                   preferred_element_type=jnp.float32)


def _knn_body(x_ref, idx_ref, d_ref):
    s = pl.program_id(0)
    x = x_ref[0]
    sq = jnp.sum(x * x, axis=1, keepdims=True)
    xb = x.astype(_BF)
    dotm = jax.lax.dot_general(xb, xb, (((1,), (1,)), ((), ())),
                               preferred_element_type=jnp.float32)
    d_ref[...] = (sq + sq.reshape(1, N)) - 2.0 * dotm
    lane = jax.lax.broadcasted_iota(jnp.int32, (N, N), 1)
    col = jax.lax.broadcasted_iota(jnp.int32, (N, 32), 1)
    m0 = jnp.min(d_ref[...], axis=1, keepdims=True)

    def body(k, carry):
        m, am_last = carry
        dcur = d_ref[...]
        eq = dcur == m
        avail = jnp.logical_and(eq, lane > am_last)
        am = jnp.min(jnp.where(avail, lane, N), axis=1, keepdims=True)
        m2 = jnp.min(jnp.where(dcur > m, dcur, jnp.inf), axis=1,
                     keepdims=True)
        cnt = jnp.sum(avail.astype(jnp.float32), axis=1, keepdims=True)
        idx_ref[0] = jnp.where(col == k, am + s * N, idx_ref[0])
        more_ties = cnt > 1.0
        m_next = jnp.where(more_ties, m, m2)
        am_next = jnp.where(more_ties, am, -1)
        return m_next, am_next

    jax.lax.fori_loop(0, K, body, (m0, jnp.full((N, 1), -1, jnp.int32)))


def _knn(x):
    C = x.shape[-1]
    return pl.pallas_call(
        _knn_body,
        grid=(S,),
        in_specs=[pl.BlockSpec((1, N, C), lambda s: (s, 0, 0))],
        out_specs=pl.BlockSpec((1, N, 32), lambda s: (s, 0, 0)),
        out_shape=jax.ShapeDtypeStruct((S, N, 32), jnp.int32),
        scratch_shapes=[pltpu.VMEM((N, N), jnp.float32)],
    )(x)


def _sc_gather(table, idx_flat):
    Cp = table.shape[-1]
    NW = 32
    b_per_w = NE // NW
    n_ch = b_per_w // _GW
    mesh = plsc.VectorSubcoreMesh(core_axis_name="c", subcore_axis_name="s")

    @functools.partial(
        pl.kernel, mesh=mesh,
        out_type=jax.ShapeDtypeStruct((NE, Cp), jnp.float32),
        scratch_types=[
            pltpu.VMEM((_GW,), jnp.int32),
            pltpu.VMEM((_GW, Cp), jnp.float32),
            pltpu.SemaphoreType.DMA,
        ])
    def k(tab_hbm, i_hbm, o_hbm, idx_v, rows_v, sem):
        wid = jax.lax.axis_index("s") * 2 + jax.lax.axis_index("c")
        base = wid * b_per_w

        @pl.loop(0, n_ch)
        def _(c):
            off = base + c * _GW
            pltpu.sync_copy(i_hbm.at[pl.ds(off, _GW)], idx_v)
            pltpu.async_copy(tab_hbm.at[idx_v], rows_v, sem).wait()
            pltpu.sync_copy(rows_v, o_hbm.at[pl.ds(off, _GW)])

    return k(table, idx_flat)


def _edge_mlp_body(xg_ref, xi_ref, w0a_ref, w0b_ref, w1_ref, w2_ref,
                   b0_ref, b1_ref, b2_ref, g0_ref, be0_ref, g1_ref, be1_ref,
                   y_ref, acc_ref):
    ph = pl.program_id(0)
    b = pl.program_id(1)
    nb = y_ref.shape[0]

    @pl.when(jnp.logical_and(ph == 0, b == 0))
    def _():
        acc_ref[0:2] = jnp.zeros((2, 32), jnp.float32)

    @pl.when(jnp.logical_and(ph == 1, b == 0))
    def _():
        acc_ref[2:4] = jnp.zeros((2, 32), jnp.float32)

    def compute_z1():
        xi = xi_ref[...]
        xg = xg_ref[...]
        diff = (xg - xi[None]).reshape(K * nb, xi.shape[-1])
        zi = _dot(xi, w0a_ref[...])
        zd = _dot(diff, w0b_ref[...])
        z1 = zd.reshape(K, nb, 32) + zi[None] + b0_ref[...][None]
        return z1.reshape(K * nb, 32)

    def norm(z, srow, ssrow, g_ref, be_ref):
        m = srow / NE
        v = ssrow / NE - m * m
        return (z - m) / jnp.sqrt(v + EPS) * g_ref[...] + be_ref[...]

    def compute_z2():
        z1 = compute_z1()
        h1 = jax.nn.relu(norm(z1, acc_ref[0:1], acc_ref[1:2],
                              g0_ref, be0_ref))
        return _dot(h1, w1_ref[...]) + b1_ref[...]

    @pl.when(ph == 0)
    def _():
        z1 = compute_z1()
        acc_ref[0:1] += jnp.sum(z1, axis=0, keepdims=True)
        acc_ref[1:2] += jnp.sum(z1 * z1, axis=0, keepdims=True)

    @pl.when(ph == 1)
    def _():
        z2 = compute_z2()
        acc_ref[2:3] += jnp.sum(z2, axis=0, keepdims=True)
        acc_ref[3:4] += jnp.sum(z2 * z2, axis=0, keepdims=True)

    @pl.when(ph == 2)
    def _():
        z2 = compute_z2()
        h2 = jax.nn.relu(norm(z2, acc_ref[2:3], acc_ref[3:4],
                              g1_ref, be1_ref))
        f = _dot(h2, w2_ref[...]) + b2_ref[...]
        mx = f[0:nb]
        for k in range(1, K):
            mx = jnp.maximum(mx, f[k * nb:(k + 1) * nb])
        y_ref[...] = mx


def _edge_mlp(xg3, xflat, p, name, C):
    Cp = xflat.shape[-1]
    W0 = p[f'{name}_W0']
    padz = jnp.zeros((Cp - C, 32), jnp.float32)
    w0a = jnp.concatenate([W0[:C], padz], axis=0)
    w0b = jnp.concatenate([W0[C:], padz], axis=0)
    row = lambda a: a.reshape(1, -1)
    nblk = NTOT // _NB
    full = lambda ph, b: (0, 0)
    return pl.pallas_call(
        _edge_mlp_body,
        grid=(3, nblk),
        in_specs=[
            pl.BlockSpec((K, _NB, Cp), lambda ph, b: (0, b, 0)),
            pl.BlockSpec((_NB, Cp), lambda ph, b: (b, 0)),
            pl.BlockSpec((Cp, 32), full),
            pl.BlockSpec((Cp, 32), full),
            pl.BlockSpec((32, 32), full),
            pl.BlockSpec((32, 32), full),
        ] + [pl.BlockSpec((1, 32), full)] * 7,
        out_specs=pl.BlockSpec((_NB, 32), lambda ph, b: (b, 0)),
        out_shape=jax.ShapeDtypeStruct((NTOT, 32), jnp.float32),
        scratch_shapes=[pltpu.VMEM((8, 32), jnp.float32)],
    )(xg3, xflat, w0a, w0b, p[f'{name}_W1'], p[f'{name}_W2'],
      row(p[f'{name}_b0']), row(p[f'{name}_b1']), row(p[f'{name}_b2']),
      row(p[f'{name}_g0']), row(p[f'{name}_be0']),
      row(p[f'{name}_g1']), row(p[f'{name}_be1']))


def _pool_body(y1_ref, y2_ref, y3_ref, w_ref, b_ref, out_ref):
    s = pl.program_id(0)
    w = w_ref[...]
    x4 = (_dot(y1_ref[...], w[0:32]) + _dot(y2_ref[...], w[32:64])
          + _dot(y3_ref[...], w[64:96]) + b_ref[...])
    out_ref[pl.ds(s, 1), :] = jnp.max(x4, axis=0, keepdims=True)


def _pool(y1, y2, y3, w, bias):
    blk = lambda s: (s, 0)
    full = lambda s: (0, 0)
    return pl.pallas_call(
        _pool_body,
        grid=(S,),
        in_specs=[pl.BlockSpec((N, 32), blk)] * 3 + [
            pl.BlockSpec((96, 1024), full),
            pl.BlockSpec((1, 1024), full),
        ],
        out_specs=pl.BlockSpec((S, 1024), full),
        out_shape=jax.ShapeDtypeStruct((S, 1024), jnp.float32),
    )(y1, y2, y3, w, bias.reshape(1, 1024))


def _head_body(pool_ref, w0, b0, w1, b1, w2, b2, w3, b3, o_ref):
    pooled = pool_ref[...]
    x5a = jnp.mean(pooled[0:T], axis=0, keepdims=True)
    x5b = jnp.mean(pooled[T:2 * T], axis=0, keepdims=True)
    y = jnp.concatenate([x5a, x5b], axis=0)
    y = jax.nn.relu(_dot(y, w0[...]) + b0[...])
    y = jax.nn.relu(_dot(y, w1[...]) + b1[...])
    y = jax.nn.relu(_dot(y, w2[...]) + b2[...])
    o_ref[...] = _dot(y, w3[...]) + b3[...]


def _head(pooled, p):
    row = lambda a: a.reshape(1, -1)
    return pl.pallas_call(
        _head_body,
        out_shape=jax.ShapeDtypeStruct((B, 10), jnp.float32),
    )(pooled, p['o_W0'], row(p['o_b0']), p['o_W1'], row(p['o_b1']),
      p['o_W2'], row(p['o_b2']), p['o_W3'], row(p['o_b3']))


def kernel(point_cloud, frame_signals, params):
    p = params
    fs = jnp.broadcast_to(frame_signals[:, :, None, :], (B, T, N, FS_DIM))
    x = jnp.concatenate([point_cloud, fs], axis=-1).reshape(S, N, IN_CH)
    ys = []
    for name in ('c1', 'c2', 'c3'):
        C = x.shape[-1]
        idx = _knn(x)
        idx_t = idx[:, :, :K].transpose(2, 0, 1).reshape(NE)
        xp = jnp.concatenate(
            [x, jnp.zeros((S, N, 128 - C), jnp.float32)], axis=-1)
        xflat = xp.reshape(NTOT, 128)
        xg = _sc_gather(xflat, idx_t).reshape(K, NTOT, 128)
        y = _edge_mlp(xg, xflat, p, name, C)
        ys.append(y)
        x = y.reshape(S, N, 32)
    pooled = _pool(ys[0], ys[1], ys[2], p['l1_W'], p['l1_b'])
    return _head(pooled, p)


# speedup vs baseline: 1.3702x; 1.3702x over previous
import functools

import jax
import jax.numpy as jnp
from jax.experimental import pallas as pl
from jax.experimental.pallas import tpu as pltpu
from jax.experimental.pallas import tpu_sc as plsc

B, T, N, K = 2, 8, 1024, 30
PC_DIM, FS_DIM = 3, 12
IN_CH = PC_DIM + FS_DIM
S = B * T
NTOT = S * N
NE = NTOT * K
EPS = 1e-5

_BF = jnp.bfloat16
_NB = 1024
_GW = 768


def _dot(a, b):
    return jnp.dot(a.astype(_BF), b.astype(_BF),
                   preferred_element_type=jnp.float32)


def _knn_body(x_ref, idx_ref, d_ref):
    s = pl.program_id(0)
    x = x_ref[0]
    sq = jnp.sum(x * x, axis=1, keepdims=True)
    xb = x.astype(_BF)
    dotm = jax.lax.dot_general(xb, xb, (((1,), (1,)), ((), ())),
                               preferred_element_type=jnp.float32)
    d_ref[...] = (sq + sq.reshape(1, N)) - 2.0 * dotm
    lane = jax.lax.broadcasted_iota(jnp.int32, (N, N), 1)
    col = jax.lax.broadcasted_iota(jnp.int32, (N, 32), 1)
    m0 = jnp.min(d_ref[...], axis=1, keepdims=True)

    def body(k, m):
        dcur = d_ref[...]
        am = jnp.min(jnp.where(dcur == m, lane, N), axis=1, keepdims=True)
        dnew = jnp.where(lane == am, jnp.inf, dcur)
        d_ref[...] = dnew
        idx_ref[0] = jnp.where(col == k, am + s * N, idx_ref[0])
        return jnp.min(dnew, axis=1, keepdims=True)

    jax.lax.fori_loop(0, K, body, m0)


def _knn(x):
    C = x.shape[-1]
    return pl.pallas_call(
        _knn_body,
        grid=(S,),
        in_specs=[pl.BlockSpec((1, N, C), lambda s: (s, 0, 0))],
        out_specs=pl.BlockSpec((1, N, 32), lambda s: (s, 0, 0)),
        out_shape=jax.ShapeDtypeStruct((S, N, 32), jnp.int32),
        scratch_shapes=[pltpu.VMEM((N, N), jnp.float32)],
    )(x)


def _sc_gather(table, idx_flat):
    Cp = table.shape[-1]
    NW = 32
    b_per_w = NE // NW
    n_ch = b_per_w // _GW
    mesh = plsc.VectorSubcoreMesh(core_axis_name="c", subcore_axis_name="s")

    @functools.partial(
        pl.kernel, mesh=mesh,
        out_type=jax.ShapeDtypeStruct((NE, Cp), jnp.float32),
        scratch_types=[
            pltpu.VMEM((_GW,), jnp.int32),
            pltpu.VMEM((_GW, Cp), jnp.float32),
            pltpu.SemaphoreType.DMA,
        ])
    def k(tab_hbm, i_hbm, o_hbm, idx_v, rows_v, sem):
        wid = jax.lax.axis_index("s") * 2 + jax.lax.axis_index("c")
        base = wid * b_per_w

        @pl.loop(0, n_ch)
        def _(c):
            off = base + c * _GW
            pltpu.sync_copy(i_hbm.at[pl.ds(off, _GW)], idx_v)
            pltpu.async_copy(tab_hbm.at[idx_v], rows_v, sem).wait()
            pltpu.sync_copy(rows_v, o_hbm.at[pl.ds(off, _GW)])

    return k(table, idx_flat)


def _edge_mlp_body(xg_ref, xi_ref, w0a_ref, w0b_ref, w1_ref, w2_ref,
                   b0_ref, b1_ref, b2_ref, g0_ref, be0_ref, g1_ref, be1_ref,
                   y_ref, acc_ref):
    ph = pl.program_id(0)
    b = pl.program_id(1)
    nb = y_ref.shape[0]

    @pl.when(jnp.logical_and(ph == 0, b == 0))
    def _():
        acc_ref[0:2] = jnp.zeros((2, 32), jnp.float32)

    @pl.when(jnp.logical_and(ph == 1, b == 0))
    def _():
        acc_ref[2:4] = jnp.zeros((2, 32), jnp.float32)

    def compute_z1():
        xi = xi_ref[...]
        xg = xg_ref[...]
        diff = (xg - xi[None]).reshape(K * nb, xi.shape[-1])
        zi = _dot(xi, w0a_ref[...])
        zd = _dot(diff, w0b_ref[...])
        z1 = zd.reshape(K, nb, 32) + zi[None] + b0_ref[...][None]
        return z1.reshape(K * nb, 32)

    def norm(z, srow, ssrow, g_ref, be_ref):
        m = srow / NE
        v = ssrow / NE - m * m
        return (z - m) / jnp.sqrt(v + EPS) * g_ref[...] + be_ref[...]

    def compute_z2():
        z1 = compute_z1()
        h1 = jax.nn.relu(norm(z1, acc_ref[0:1], acc_ref[1:2],
                              g0_ref, be0_ref))
        return _dot(h1, w1_ref[...]) + b1_ref[...]

    @pl.when(ph == 0)
    def _():
        z1 = compute_z1()
        acc_ref[0:1] += jnp.sum(z1, axis=0, keepdims=True)
        acc_ref[1:2] += jnp.sum(z1 * z1, axis=0, keepdims=True)

    @pl.when(ph == 1)
    def _():
        z2 = compute_z2()
        acc_ref[2:3] += jnp.sum(z2, axis=0, keepdims=True)
        acc_ref[3:4] += jnp.sum(z2 * z2, axis=0, keepdims=True)

    @pl.when(ph == 2)
    def _():
        z2 = compute_z2()
        h2 = jax.nn.relu(norm(z2, acc_ref[2:3], acc_ref[3:4],
                              g1_ref, be1_ref))
        f = _dot(h2, w2_ref[...]) + b2_ref[...]
        mx = f[0:nb]
        for k in range(1, K):
            mx = jnp.maximum(mx, f[k * nb:(k + 1) * nb])
        y_ref[...] = mx


def _edge_mlp(xg3, xflat, p, name, C):
    Cp = xflat.shape[-1]
    W0 = p[f'{name}_W0']
    padz = jnp.zeros((Cp - C, 32), jnp.float32)
    w0a = jnp.concatenate([W0[:C], padz], axis=0)
    w0b = jnp.concatenate([W0[C:], padz], axis=0)
    row = lambda a: a.reshape(1, -1)
    nblk = NTOT // _NB
    full = lambda ph, b: (0, 0)
    return pl.pallas_call(
        _edge_mlp_body,
        grid=(3, nblk),
        in_specs=[
            pl.BlockSpec((K, _NB, Cp), lambda ph, b: (0, b, 0)),
            pl.BlockSpec((_NB, Cp), lambda ph, b: (b, 0)),
            pl.BlockSpec((Cp, 32), full),
            pl.BlockSpec((Cp, 32), full),
            pl.BlockSpec((32, 32), full),
            pl.BlockSpec((32, 32), full),
        ] + [pl.BlockSpec((1, 32), full)] * 7,
        out_specs=pl.BlockSpec((_NB, 32), lambda ph, b: (b, 0)),
        out_shape=jax.ShapeDtypeStruct((NTOT, 32), jnp.float32),
        scratch_shapes=[pltpu.VMEM((8, 32), jnp.float32)],
    )(xg3, xflat, w0a, w0b, p[f'{name}_W1'], p[f'{name}_W2'],
      row(p[f'{name}_b0']), row(p[f'{name}_b1']), row(p[f'{name}_b2']),
      row(p[f'{name}_g0']), row(p[f'{name}_be0']),
      row(p[f'{name}_g1']), row(p[f'{name}_be1']))


def _pool_body(y1_ref, y2_ref, y3_ref, w_ref, b_ref, out_ref):
    s = pl.program_id(0)
    w = w_ref[...]
    x4 = (_dot(y1_ref[...], w[0:32]) + _dot(y2_ref[...], w[32:64])
          + _dot(y3_ref[...], w[64:96]) + b_ref[...])
    out_ref[pl.ds(s, 1), :] = jnp.max(x4, axis=0, keepdims=True)


def _pool(y1, y2, y3, w, bias):
    blk = lambda s: (s, 0)
    full = lambda s: (0, 0)
    return pl.pallas_call(
        _pool_body,
        grid=(S,),
        in_specs=[pl.BlockSpec((N, 32), blk)] * 3 + [
            pl.BlockSpec((96, 1024), full),
            pl.BlockSpec((1, 1024), full),
        ],
        out_specs=pl.BlockSpec((S, 1024), full),
        out_shape=jax.ShapeDtypeStruct((S, 1024), jnp.float32),
    )(y1, y2, y3, w, bias.reshape(1, 1024))


def _head_body(pool_ref, w0, b0, w1, b1, w2, b2, w3, b3, o_ref):
    pooled = pool_ref[...]
    x5a = jnp.mean(pooled[0:T], axis=0, keepdims=True)
    x5b = jnp.mean(pooled[T:2 * T], axis=0, keepdims=True)
    y = jnp.concatenate([x5a, x5b], axis=0)
    y = jax.nn.relu(_dot(y, w0[...]) + b0[...])
    y = jax.nn.relu(_dot(y, w1[...]) + b1[...])
    y = jax.nn.relu(_dot(y, w2[...]) + b2[...])
    o_ref[...] = _dot(y, w3[...]) + b3[...]


def _head(pooled, p):
    row = lambda a: a.reshape(1, -1)
    return pl.pallas_call(
        _head_body,
        out_shape=jax.ShapeDtypeStruct((B, 10), jnp.float32),
    )(pooled, p['o_W0'], row(p['o_b0']), p['o_W1'], row(p['o_b1']),
      p['o_W2'], row(p['o_b2']), p['o_W3'], row(p['o_b3']))


def kernel(point_cloud, frame_signals, params):
    p = params
    fs = jnp.broadcast_to(frame_signals[:, :, None, :], (B, T, N, FS_DIM))
    x = jnp.concatenate([point_cloud, fs], axis=-1).reshape(S, N, IN_CH)
    ys = []
    for name in ('c1', 'c2', 'c3'):
        C = x.shape[-1]
        idx = _knn(x)
        idx_t = idx[:, :, :K].transpose(2, 0, 1).reshape(NE)
        xp = jnp.concatenate(
            [x, jnp.zeros((S, N, 128 - C), jnp.float32)], axis=-1)
        xflat = xp.reshape(NTOT, 128)
        xg = _sc_gather(xflat, idx_t).reshape(K, NTOT, 128)
        y = _edge_mlp(xg, xflat, p, name, C)
        ys.append(y)
        x = y.reshape(S, N, 32)
    pooled = _pool(ys[0], ys[1], ys[2], p['l1_W'], p['l1_b'])
    return _head(pooled, p)


# speedup vs baseline: 1.3802x; 1.0072x over previous
import functools

import jax
import jax.numpy as jnp
from jax.experimental import pallas as pl
from jax.experimental.pallas import tpu as pltpu
from jax.experimental.pallas import tpu_sc as plsc

B, T, N, K = 2, 8, 1024, 30
PC_DIM, FS_DIM = 3, 12
IN_CH = PC_DIM + FS_DIM
S = B * T
NTOT = S * N
NE = NTOT * K
EPS = 1e-5

_BF = jnp.bfloat16
_NB = 1024
_GW = 384


def _dot(a, b):
    return jnp.dot(a.astype(_BF), b.astype(_BF),
                   preferred_element_type=jnp.float32)


def _knn_body(x_ref, idx_ref, d_ref):
    s = pl.program_id(0)
    x = x_ref[0]
    sq = jnp.sum(x * x, axis=1, keepdims=True)
    xb = x.astype(_BF)
    dotm = jax.lax.dot_general(xb, xb, (((1,), (1,)), ((), ())),
                               preferred_element_type=jnp.float32)
    d_ref[...] = (sq + sq.reshape(1, N)) - 2.0 * dotm
    lane = jax.lax.broadcasted_iota(jnp.int32, (N, N), 1)
    col = jax.lax.broadcasted_iota(jnp.int32, (N, 32), 1)
    m0 = jnp.min(d_ref[...], axis=1, keepdims=True)

    def body(k, m):
        dcur = d_ref[...]
        am = jnp.min(jnp.where(dcur == m, lane, N), axis=1, keepdims=True)
        dnew = jnp.where(lane == am, jnp.inf, dcur)
        d_ref[...] = dnew
        idx_ref[0] = jnp.where(col == k, am + s * N, idx_ref[0])
        return jnp.min(dnew, axis=1, keepdims=True)

    jax.lax.fori_loop(0, K, body, m0)


def _knn(x):
    C = x.shape[-1]
    return pl.pallas_call(
        _knn_body,
        grid=(S,),
        in_specs=[pl.BlockSpec((1, N, C), lambda s: (s, 0, 0))],
        out_specs=pl.BlockSpec((1, N, 32), lambda s: (s, 0, 0)),
        out_shape=jax.ShapeDtypeStruct((S, N, 32), jnp.int32),
        scratch_shapes=[pltpu.VMEM((N, N), jnp.float32)],
    )(x)


def _sc_gather(table, idx_flat):
    Cp = table.shape[-1]
    NW = 32
    b_per_w = NE // NW
    n_ch = b_per_w // _GW
    mesh = plsc.VectorSubcoreMesh(core_axis_name="c", subcore_axis_name="s")

    @functools.partial(
        pl.kernel, mesh=mesh,
        out_type=jax.ShapeDtypeStruct((NE, Cp), jnp.float32),
        scratch_types=[
            pltpu.VMEM((_GW,), jnp.int32),
            pltpu.VMEM((_GW,), jnp.int32),
            pltpu.VMEM((_GW, Cp), jnp.float32),
            pltpu.VMEM((_GW, Cp), jnp.float32),
            pltpu.SemaphoreType.DMA,
            pltpu.SemaphoreType.DMA,
        ])
    def k(tab_hbm, i_hbm, o_hbm, i0, i1, r0, r1, sg0, sg1):
        wid = jax.lax.axis_index("s") * 2 + jax.lax.axis_index("c")
        base = wid * b_per_w

        def load_idx(cc, iv):
            pltpu.sync_copy(i_hbm.at[pl.ds(base + cc * _GW, _GW)], iv)

        load_idx(0, i0)
        pltpu.async_copy(tab_hbm.at[i0], r0, sg0)

        @pl.loop(0, n_ch, step=2)
        def _(c):
            load_idx(c + 1, i1)
            pltpu.async_copy(tab_hbm.at[i1], r1, sg1)
            pltpu.make_async_copy(tab_hbm.at[i0], r0, sg0).wait()
            pltpu.sync_copy(r0, o_hbm.at[pl.ds(base + c * _GW, _GW)])

            @pl.when(c + 2 < n_ch)
            def _():
                load_idx(c + 2, i0)
                pltpu.async_copy(tab_hbm.at[i0], r0, sg0)

            pltpu.make_async_copy(tab_hbm.at[i1], r1, sg1).wait()
            pltpu.sync_copy(r1, o_hbm.at[pl.ds(base + (c + 1) * _GW, _GW)])

    return k(table, idx_flat)


def _edge_mlp_body(xg_ref, xi_ref, w0a_ref, w0b_ref, w1_ref, w2_ref,
                   b0_ref, b1_ref, b2_ref, g0_ref, be0_ref, g1_ref, be1_ref,
                   y_ref, acc_ref):
    ph = pl.program_id(0)
    b = pl.program_id(1)
    nb = y_ref.shape[0]

    @pl.when(jnp.logical_and(ph == 0, b == 0))
    def _():
        acc_ref[0:2] = jnp.zeros((2, 32), jnp.float32)

    @pl.when(jnp.logical_and(ph == 1, b == 0))
    def _():
        acc_ref[2:4] = jnp.zeros((2, 32), jnp.float32)

    def compute_z1():
        xi = xi_ref[...]
        xg = xg_ref[...]
        diff = (xg - xi[None]).reshape(K * nb, xi.shape[-1])
        zi = _dot(xi, w0a_ref[...])
        zd = _dot(diff, w0b_ref[...])
        z1 = zd.reshape(K, nb, 32) + zi[None] + b0_ref[...][None]
        return z1.reshape(K * nb, 32)

    def norm(z, srow, ssrow, g_ref, be_ref):
        m = srow / NE
        v = ssrow / NE - m * m
        return (z - m) / jnp.sqrt(v + EPS) * g_ref[...] + be_ref[...]

    def compute_z2():
        z1 = compute_z1()
        h1 = jax.nn.relu(norm(z1, acc_ref[0:1], acc_ref[1:2],
                              g0_ref, be0_ref))
        return _dot(h1, w1_ref[...]) + b1_ref[...]

    @pl.when(ph == 0)
    def _():
        z1 = compute_z1()
        acc_ref[0:1] += jnp.sum(z1, axis=0, keepdims=True)
        acc_ref[1:2] += jnp.sum(z1 * z1, axis=0, keepdims=True)

    @pl.when(ph == 1)
    def _():
        z2 = compute_z2()
        acc_ref[2:3] += jnp.sum(z2, axis=0, keepdims=True)
        acc_ref[3:4] += jnp.sum(z2 * z2, axis=0, keepdims=True)

    @pl.when(ph == 2)
    def _():
        z2 = compute_z2()
        h2 = jax.nn.relu(norm(z2, acc_ref[2:3], acc_ref[3:4],
                              g1_ref, be1_ref))
        f = _dot(h2, w2_ref[...]) + b2_ref[...]
        mx = f[0:nb]
        for k in range(1, K):
            mx = jnp.maximum(mx, f[k * nb:(k + 1) * nb])
        y_ref[...] = mx


def _edge_mlp(xg3, xflat, p, name, C):
    Cp = xflat.shape[-1]
    W0 = p[f'{name}_W0']
    padz = jnp.zeros((Cp - C, 32), jnp.float32)
    w0a = jnp.concatenate([W0[:C], padz], axis=0)
    w0b = jnp.concatenate([W0[C:], padz], axis=0)
    row = lambda a: a.reshape(1, -1)
    nblk = NTOT // _NB
    full = lambda ph, b: (0, 0)
    return pl.pallas_call(
        _edge_mlp_body,
        grid=(3, nblk),
        in_specs=[
            pl.BlockSpec((K, _NB, Cp), lambda ph, b: (0, b, 0)),
            pl.BlockSpec((_NB, Cp), lambda ph, b: (b, 0)),
            pl.BlockSpec((Cp, 32), full),
            pl.BlockSpec((Cp, 32), full),
            pl.BlockSpec((32, 32), full),
            pl.BlockSpec((32, 32), full),
        ] + [pl.BlockSpec((1, 32), full)] * 7,
        out_specs=pl.BlockSpec((_NB, 32), lambda ph, b: (b, 0)),
        out_shape=jax.ShapeDtypeStruct((NTOT, 32), jnp.float32),
        scratch_shapes=[pltpu.VMEM((8, 32), jnp.float32)],
    )(xg3, xflat, w0a, w0b, p[f'{name}_W1'], p[f'{name}_W2'],
      row(p[f'{name}_b0']), row(p[f'{name}_b1']), row(p[f'{name}_b2']),
      row(p[f'{name}_g0']), row(p[f'{name}_be0']),
      row(p[f'{name}_g1']), row(p[f'{name}_be1']))


def _pool_body(y1_ref, y2_ref, y3_ref, w_ref, b_ref, out_ref):
    s = pl.program_id(0)
    w = w_ref[...]
    x4 = (_dot(y1_ref[...], w[0:32]) + _dot(y2_ref[...], w[32:64])
          + _dot(y3_ref[...], w[64:96]) + b_ref[...])
    out_ref[pl.ds(s, 1), :] = jnp.max(x4, axis=0, keepdims=True)


def _pool(y1, y2, y3, w, bias):
    blk = lambda s: (s, 0)
    full = lambda s: (0, 0)
    return pl.pallas_call(
        _pool_body,
        grid=(S,),
        in_specs=[pl.BlockSpec((N, 32), blk)] * 3 + [
            pl.BlockSpec((96, 1024), full),
            pl.BlockSpec((1, 1024), full),
        ],
        out_specs=pl.BlockSpec((S, 1024), full),
        out_shape=jax.ShapeDtypeStruct((S, 1024), jnp.float32),
    )(y1, y2, y3, w, bias.reshape(1, 1024))


def _head_body(pool_ref, w0, b0, w1, b1, w2, b2, w3, b3, o_ref):
    pooled = pool_ref[...]
    x5a = jnp.mean(pooled[0:T], axis=0, keepdims=True)
    x5b = jnp.mean(pooled[T:2 * T], axis=0, keepdims=True)
    y = jnp.concatenate([x5a, x5b], axis=0)
    y = jax.nn.relu(_dot(y, w0[...]) + b0[...])
    y = jax.nn.relu(_dot(y, w1[...]) + b1[...])
    y = jax.nn.relu(_dot(y, w2[...]) + b2[...])
    o_ref[...] = _dot(y, w3[...]) + b3[...]


def _head(pooled, p):
    row = lambda a: a.reshape(1, -1)
    return pl.pallas_call(
        _head_body,
        out_shape=jax.ShapeDtypeStruct((B, 10), jnp.float32),
    )(pooled, p['o_W0'], row(p['o_b0']), p['o_W1'], row(p['o_b1']),
      p['o_W2'], row(p['o_b2']), p['o_W3'], row(p['o_b3']))


def kernel(point_cloud, frame_signals, params):
    p = params
    fs = jnp.broadcast_to(frame_signals[:, :, None, :], (B, T, N, FS_DIM))
    x = jnp.concatenate([point_cloud, fs], axis=-1).reshape(S, N, IN_CH)
    ys = []
    for name in ('c1', 'c2', 'c3'):
        C = x.shape[-1]
        idx = _knn(x)
        idx_t = idx[:, :, :K].transpose(2, 0, 1).reshape(NE)
        xp = jnp.concatenate(
            [x, jnp.zeros((S, N, 128 - C), jnp.float32)], axis=-1)
        xflat = xp.reshape(NTOT, 128)
        xg = _sc_gather(xflat, idx_t).reshape(K, NTOT, 128)
        y = _edge_mlp(xg, xflat, p, name, C)
        ys.append(y)
        x = y.reshape(S, N, 32)
    pooled = _pool(ys[0], ys[1], ys[2], p['l1_W'], p['l1_b'])
    return _head(pooled, p)
